# baseline (speedup 1.0000x reference)
import functools

import jax
import jax.numpy as jnp
from jax import lax
from jax.experimental import pallas as pl
from jax.experimental.pallas import tpu as pltpu
from jax.experimental.pallas import tpu_sc as plsc

N_NODES = 10000
D = 128
NC = 2
NS = 16
CHUNK = 128
N_PAD = 10240
HALF = N_PAD // NC
ROWS_PER_TILE = HALF // NS
PAD_IDX = N_PAD - 1
PADC = 5
BLK = 256
N_BLOCKS = N_PAD // BLK


def _sc_route_body(g16, capw, srcf, dstf, csrc, cdst, counts,
                   src_v, dst_v, buf_s, buf_d, cnt_v):
    c = lax.axis_index("c")
    s = lax.axis_index("s")
    lo = c * HALF
    pltpu.sync_copy(srcf.at[s], src_v)
    pltpu.sync_copy(dstf.at[s], dst_v)

    pad_s = jnp.full((16,), PAD_IDX, jnp.int32)
    pad_d = jnp.zeros((16,), jnp.int32)

    def prefill(t, carry):
        buf_s[pl.ds(16 * t, 16)] = pad_s
        buf_d[pl.ds(16 * t, 16)] = pad_d
        return carry

    lax.fori_loop(0, capw // 16, prefill, 0)

    def body(g, off):
        s16 = src_v[g]
        dl = dst_v[g] - lo
        m = (dl >= 0) & (dl < HALF)
        cnt = jnp.max(plsc.all_reduce_population_count(m))
        plsc.store_compressed(buf_s.at[pl.ds(off, 16)], s16, mask=m)
        plsc.store_compressed(buf_d.at[pl.ds(off, 16)], dl, mask=m)
        return off + cnt

    off = lax.fori_loop(0, g16, body, jnp.int32(0))

    def padbody(t, carry):
        buf_s[pl.ds(off + 16 * t, 16)] = pad_s
        buf_d[pl.ds(off + 16 * t, 16)] = pad_d
        return carry

    lax.fori_loop(0, CHUNK // 16, padbody, 0)

    nch = (off + CHUNK - 1) // CHUNK
    cnt_v[...] = jnp.broadcast_to(nch, (16,)).astype(jnp.int32)
    pltpu.sync_copy(cnt_v, counts.at[s, c])
    pltpu.sync_copy(buf_s, csrc.at[c, s])
    pltpu.sync_copy(buf_d, cdst.at[c, s])


def _sc_route(srcf, dstf, g16, capw):
    mesh = plsc.VectorSubcoreMesh(core_axis_name="c", subcore_axis_name="s",
                                  num_cores=NC, num_subcores=NS)
    kern = pl.kernel(
        functools.partial(_sc_route_body, g16, capw),
        out_type=[
            jax.ShapeDtypeStruct((NC, NS, capw), jnp.int32),
            jax.ShapeDtypeStruct((NC, NS, capw), jnp.int32),
            jax.ShapeDtypeStruct((NS, NC, 16), jnp.int32),
        ],
        mesh=mesh,
        scratch_types=[
            pltpu.VMEM((g16, 16), jnp.int32),
            pltpu.VMEM((g16, 16), jnp.int32),
            pltpu.VMEM((capw,), jnp.int32),
            pltpu.VMEM((capw,), jnp.int32),
            pltpu.VMEM((16,), jnp.int32),
        ],
        compiler_params=pltpu.CompilerParams(use_tc_tiling_on_sc=False,
                                            needs_layout_passes=False),
        name="gcn_sc_route",
    )
    return kern(srcf, dstf)


def _sc_agg_body(capc, ring, table, csrc, cdst, zinit, out,
                 src_v, dst_v, rows, gsems, ssems, acc):
    c = lax.axis_index("c")
    s = lax.axis_index("s")
    r0 = s * ROWS_PER_TILE
    pltpu.sync_copy(zinit, acc.at[pl.ds(r0, ROWS_PER_TILE)])
    pltpu.sync_copy(csrc.at[c, s], src_v)
    pltpu.sync_copy(cdst.at[c, s], dst_v)
    plsc.subcore_barrier()

    def gather(idx, b):
        pltpu.async_copy(table.at[src_v.at[idx]], rows.at[b], gsems.at[b])

    def wait_gather(idx, b):
        pltpu.make_async_copy(table.at[src_v.at[idx]], rows.at[b],
                              gsems.at[b]).wait()

    def scatter(idx, b):
        pltpu.async_copy(rows.at[b], acc.at[dst_v.at[idx]], ssems.at[b],
                         add=True)

    def wait_scatter(idx, b):
        pltpu.make_async_copy(rows.at[b], acc.at[dst_v.at[idx]],
                              ssems.at[b]).wait()

    for b in range(ring):
        gather(b, b)

    def body(j, carry):
        for b in range(ring):
            idx = ring * j + b
            wait_gather(idx, b)
            scatter(idx, b)
            wait_scatter(idx, b)
            gather(idx + ring, b)
        return carry

    lax.fori_loop(0, capc // ring - 1, body, 0)
    for b in range(ring):
        idx = capc - ring + b
        wait_gather(idx, b)
        scatter(idx, b)
        wait_scatter(idx, b)
    plsc.subcore_barrier()
    pltpu.sync_copy(acc.at[pl.ds(r0, ROWS_PER_TILE)],
                    out.at[c, pl.ds(r0, ROWS_PER_TILE)])


def _sc_aggregate(table, csrc, cdst, zinit, d, capc):
    ring = 3 if d <= 16 else 2
    mesh = plsc.VectorSubcoreMesh(core_axis_name="c", subcore_axis_name="s",
                                  num_cores=NC, num_subcores=NS)
    kern = pl.kernel(
        functools.partial(_sc_agg_body, capc, ring),
        out_type=jax.ShapeDtypeStruct((NC, HALF, d), jnp.float32),
        mesh=mesh,
        scratch_types=[
            pltpu.VMEM((capc, CHUNK), jnp.int32),
            pltpu.VMEM((capc, CHUNK), jnp.int32),
            pltpu.VMEM((ring, CHUNK, d), jnp.float32),
            pltpu.SemaphoreType.DMA((ring,)),
            pltpu.SemaphoreType.DMA((ring,)),
            pltpu.VMEM_SHARED((HALF, d), jnp.float32),
        ],
        compiler_params=pltpu.CompilerParams(use_tc_tiling_on_sc=False),
        name=f"gcn_sc_agg_d{d}",
    )
    return kern(table, csrc, cdst, zinit)


def _tc_first_body(x_ref, w_ref, degp_ref, hs_ref, dinv_ref):
    i = pl.program_id(0)
    deg = jnp.sum(degp_ref[...], axis=1) + 1.0
    row = i * BLK + lax.broadcasted_iota(jnp.int32, (BLK,), 0)
    dinv = jnp.where(row < N_NODES, lax.rsqrt(deg), 0.0)
    dinv_b = jnp.broadcast_to(dinv[:, None], (BLK, D))
    dinv_ref[...] = dinv_b
    h = jnp.dot(x_ref[...], w_ref[...], preferred_element_type=jnp.float32)
    hs_ref[...] = h * dinv_b


def _tc_first(x_pad, w1, degs):
    return pl.pallas_call(
        _tc_first_body,
        grid=(N_BLOCKS,),
        in_specs=[
            pl.BlockSpec((BLK, D), lambda i: (i, 0)),
            pl.BlockSpec((D, D), lambda i: (0, 0)),
            pl.BlockSpec((BLK, 16), lambda i: (i, 0)),
        ],
        out_specs=[
            pl.BlockSpec((BLK, D), lambda i: (i, 0)),
            pl.BlockSpec((BLK, D), lambda i: (i, 0)),
        ],
        out_shape=[
            jax.ShapeDtypeStruct((N_PAD, D), jnp.float32),
            jax.ShapeDtypeStruct((N_PAD, D), jnp.float32),
        ],
        name="gcn_tc_first",
    )(x_pad, w1, degs)


def _tc_mid_body(agg_ref, hs_ref, dinv_ref, b_ref, w_ref, pre_ref, o_ref):
    tot = agg_ref[...] + hs_ref[...]
    pre = tot * dinv_ref[...] + b_ref[...]
    pre_ref[...] = pre
    act = jnp.maximum(pre, 0.0)
    o_ref[...] = jnp.dot(act, w_ref[...],
                         preferred_element_type=jnp.float32) * dinv_ref[...]


def _tc_mid(agg, hs, dinv_b, b, w_next):
    return pl.pallas_call(
        _tc_mid_body,
        grid=(N_BLOCKS,),
        in_specs=[
            pl.BlockSpec((BLK, D), lambda i: (i, 0)),
            pl.BlockSpec((BLK, D), lambda i: (i, 0)),
            pl.BlockSpec((BLK, D), lambda i: (i, 0)),
            pl.BlockSpec((1, D), lambda i: (0, 0)),
            pl.BlockSpec((D, D), lambda i: (0, 0)),
        ],
        out_specs=[
            pl.BlockSpec((BLK, D), lambda i: (i, 0)),
            pl.BlockSpec((BLK, D), lambda i: (i, 0)),
        ],
        out_shape=[
            jax.ShapeDtypeStruct((N_PAD, D), jnp.float32),
            jax.ShapeDtypeStruct((N_PAD, D), jnp.float32),
        ],
        name="gcn_tc_mid",
    )(agg, hs, dinv_b, b.reshape(1, D), w_next)


def kernel(x, edge_index, W1, b1, W2, b2, W3, b3):
    e = edge_index.shape[1]
    nchunk = -(-e // (NS * CHUNK))
    g16 = nchunk * CHUNK // 16
    capc = -(-(nchunk + 1) // 6) * 6
    capw = capc * CHUNK
    e_pad = NS * nchunk * CHUNK
    src = edge_index[0].astype(jnp.int32)
    dst = edge_index[1].astype(jnp.int32)
    fill_s = jnp.full((e_pad - e,), PAD_IDX, jnp.int32)
    srcf = jnp.concatenate([src, fill_s]).reshape(NS, g16, 16)
    dstf = jnp.concatenate([dst, fill_s]).reshape(NS, g16, 16)

    x_pad = jnp.pad(x, ((0, N_PAD - N_NODES), (0, 0)))
    e0_table = jnp.zeros((N_PAD, 16), jnp.float32).at[:N_NODES, 0].set(1.0)
    z16 = jnp.zeros((ROWS_PER_TILE, 16), jnp.float32)
    z128 = jnp.zeros((ROWS_PER_TILE, D), jnp.float32)

    csrc_f, cdst_f, counts = _sc_route(srcf, dstf, g16, capw)
    csrc = csrc_f.reshape(NC, NS, capc, CHUNK)
    cdst = cdst_f.reshape(NC, NS, capc, CHUNK)

    degs = _sc_aggregate(e0_table, csrc, cdst, z16, 16, capc)
    hs1, dinv_b = _tc_first(x_pad, W1, degs.reshape(N_PAD, 16))

    def step(hs, wb):
        w_next, b = wb
        agg = _sc_aggregate(hs, csrc, cdst, z128, D, capc)
        pre, hs_next = _tc_mid(agg.reshape(N_PAD, D), hs, dinv_b, b, w_next)
        return hs_next, pre

    ws = jnp.stack([W2, W3, jnp.zeros_like(W3)])
    bs = jnp.stack([b1, b2, b3])
    _, pres = lax.scan(step, hs1, (ws, bs))
    return pres[2][:N_NODES]


# speedup vs baseline: 21.9657x; 21.9657x over previous
import functools

import jax
import jax.numpy as jnp
from jax import lax
from jax.experimental import pallas as pl
from jax.experimental.pallas import tpu as pltpu
from jax.experimental.pallas import tpu_sc as plsc

N_NODES = 10000
D = 128
NC = 2
NS = 16
CHUNK = 128
N_PAD = 10240
HALF = N_PAD // NC
ACC_ROWS = HALF + 8
TRASH = HALF
ROWS_PER_TILE = HALF // NS
PAD_IDX = N_PAD - 1
NBUF = 2
BLK = 256
N_BLOCKS = N_PAD // BLK


def _sc_agg_body(nchunk, table, srcs, dsts, zinit, out,
                 src_v, dst_v, rows, gsems, ssems, acc):
    c = lax.axis_index("c")
    s = lax.axis_index("s")
    r0 = s * ROWS_PER_TILE
    pltpu.sync_copy(zinit, acc.at[pl.ds(r0, ROWS_PER_TILE)])
    pltpu.sync_copy(srcs.at[s], src_v)
    pltpu.sync_copy(dsts.at[c, s], dst_v)
    plsc.subcore_barrier()

    for b in range(NBUF):
        pltpu.async_copy(table.at[src_v.at[b]], rows.at[b], gsems.at[b])

    def body(j, carry):
        for b in range(NBUF):
            idx = NBUF * j + b
            pltpu.make_async_copy(table.at[src_v.at[idx]], rows.at[b],
                                  gsems.at[b]).wait()
            pltpu.async_copy(rows.at[b], acc.at[dst_v.at[idx]], ssems.at[b],
                             add=True)
            pltpu.make_async_copy(rows.at[b], acc.at[dst_v.at[idx]],
                                  ssems.at[b]).wait()
            pltpu.async_copy(table.at[src_v.at[idx + NBUF]], rows.at[b],
                             gsems.at[b])
        return carry

    def tail(j, carry):
        for b in range(NBUF):
            idx = NBUF * j + b
            pltpu.make_async_copy(table.at[src_v.at[idx]], rows.at[b],
                                  gsems.at[b]).wait()
            pltpu.sync_copy(rows.at[b], acc.at[dst_v.at[idx]], add=True)
        return carry

    ngroups = nchunk // NBUF
    lax.fori_loop(0, ngroups - 1, body, 0)
    tail(ngroups - 1, 0)
    plsc.subcore_barrier()
    pltpu.sync_copy(acc.at[pl.ds(r0, ROWS_PER_TILE)],
                    out.at[c, pl.ds(r0, ROWS_PER_TILE)])


def _sc_aggregate(table, srcs, dsts, zinit, d, nchunk):
    mesh = plsc.VectorSubcoreMesh(core_axis_name="c", subcore_axis_name="s",
                                  num_cores=NC, num_subcores=NS)
    kern = pl.kernel(
        functools.partial(_sc_agg_body, nchunk),
        out_type=jax.ShapeDtypeStruct((NC, HALF, d), jnp.float32),
        mesh=mesh,
        scratch_types=[
            pltpu.VMEM((nchunk, CHUNK), jnp.int32),
            pltpu.VMEM((nchunk, CHUNK), jnp.int32),
            pltpu.VMEM((NBUF, CHUNK, d), jnp.float32),
            pltpu.SemaphoreType.DMA((NBUF,)),
            pltpu.SemaphoreType.DMA((NBUF,)),
            pltpu.VMEM_SHARED((ACC_ROWS, d), jnp.float32),
        ],
        compiler_params=pltpu.CompilerParams(use_tc_tiling_on_sc=False),
        name=f"gcn_sc_agg_d{d}",
    )
    return kern(table, srcs, dsts, zinit)


def _tc_first_body(x_ref, w_ref, degp_ref, hs_ref, dinv_ref):
    i = pl.program_id(0)
    deg = jnp.sum(degp_ref[...], axis=1) + 1.0
    row = i * BLK + lax.broadcasted_iota(jnp.int32, (BLK,), 0)
    dinv = jnp.where(row < N_NODES, lax.rsqrt(deg), 0.0)
    dinv_b = jnp.broadcast_to(dinv[:, None], (BLK, D))
    dinv_ref[...] = dinv_b
    h = jnp.dot(x_ref[...], w_ref[...], preferred_element_type=jnp.float32)
    hs_ref[...] = h * dinv_b


def _tc_first(x_pad, w1, degs):
    return pl.pallas_call(
        _tc_first_body,
        grid=(N_BLOCKS,),
        in_specs=[
            pl.BlockSpec((BLK, D), lambda i: (i, 0)),
            pl.BlockSpec((D, D), lambda i: (0, 0)),
            pl.BlockSpec((BLK, 16), lambda i: (i, 0)),
        ],
        out_specs=[
            pl.BlockSpec((BLK, D), lambda i: (i, 0)),
            pl.BlockSpec((BLK, D), lambda i: (i, 0)),
        ],
        out_shape=[
            jax.ShapeDtypeStruct((N_PAD, D), jnp.float32),
            jax.ShapeDtypeStruct((N_PAD, D), jnp.float32),
        ],
        name="gcn_tc_first",
    )(x_pad, w1, degs)


def _tc_mid_body(agg_ref, hs_ref, dinv_ref, b_ref, w_ref, pre_ref, o_ref):
    tot = agg_ref[...] + hs_ref[...]
    pre = tot * dinv_ref[...] + b_ref[...]
    pre_ref[...] = pre
    act = jnp.maximum(pre, 0.0)
    o_ref[...] = jnp.dot(act, w_ref[...],
                         preferred_element_type=jnp.float32) * dinv_ref[...]


def _tc_mid(agg, hs, dinv_b, b, w_next):
    return pl.pallas_call(
        _tc_mid_body,
        grid=(N_BLOCKS,),
        in_specs=[
            pl.BlockSpec((BLK, D), lambda i: (i, 0)),
            pl.BlockSpec((BLK, D), lambda i: (i, 0)),
            pl.BlockSpec((BLK, D), lambda i: (i, 0)),
            pl.BlockSpec((1, D), lambda i: (0, 0)),
            pl.BlockSpec((D, D), lambda i: (0, 0)),
        ],
        out_specs=[
            pl.BlockSpec((BLK, D), lambda i: (i, 0)),
            pl.BlockSpec((BLK, D), lambda i: (i, 0)),
        ],
        out_shape=[
            jax.ShapeDtypeStruct((N_PAD, D), jnp.float32),
            jax.ShapeDtypeStruct((N_PAD, D), jnp.float32),
        ],
        name="gcn_tc_mid",
    )(agg, hs, dinv_b, b.reshape(1, D), w_next)


def kernel(x, edge_index, W1, b1, W2, b2, W3, b3):
    e = edge_index.shape[1]
    nchunk = -(-e // (NS * CHUNK))
    nchunk = -(-nchunk // NBUF) * NBUF
    e_pad = NS * nchunk * CHUNK
    src = edge_index[0].astype(jnp.int32)
    dst = edge_index[1].astype(jnp.int32)
    fill = jnp.full((e_pad - e,), PAD_IDX, jnp.int32)
    src = jnp.concatenate([src, fill])
    dst = jnp.concatenate([dst, fill])
    srcs = src.reshape(NS, nchunk, CHUNK)
    dst_loc = dst[None, :] - jnp.array([0, HALF], jnp.int32)[:, None]
    dsts = jnp.where((dst_loc >= 0) & (dst_loc < HALF), dst_loc, TRASH)
    dsts = dsts.reshape(NC, NS, nchunk, CHUNK)

    x_pad = jnp.pad(x, ((0, N_PAD - N_NODES), (0, 0)))
    e0_table = jnp.zeros((N_PAD, 16), jnp.float32).at[:N_NODES, 0].set(1.0)
    z16 = jnp.zeros((ROWS_PER_TILE, 16), jnp.float32)
    z128 = jnp.zeros((ROWS_PER_TILE, D), jnp.float32)

    degs = _sc_aggregate(e0_table, srcs, dsts, z16, 16, nchunk)
    hs1, dinv_b = _tc_first(x_pad, W1, degs.reshape(N_PAD, 16))

    def step(hs, wb):
        w_next, b = wb
        agg = _sc_aggregate(hs, srcs, dsts, z128, D, nchunk)
        pre, hs_next = _tc_mid(agg.reshape(N_PAD, D), hs, dinv_b, b, w_next)
        return hs_next, pre

    ws = jnp.stack([W2, W3, jnp.zeros_like(W3)])
    bs = jnp.stack([b1, b2, b3])
    _, pres = lax.scan(step, hs1, (ws, bs))
    return pres[2][:N_NODES]
